# Initial kernel scaffold; baseline (speedup 1.0000x reference)
import jax
import jax.numpy as jnp
from jax.experimental import pallas as pl


def kernel(x):
    raise NotImplementedError("write your pallas kernel here")


# speedup vs baseline: 4.9496x; 4.9496x over previous
import functools

import jax
import jax.numpy as jnp
from jax import lax
from jax.experimental import pallas as pl
from jax.experimental.pallas import tpu as pltpu
from jax.experimental.pallas import tpu_sc as plsc

N_ROWS = 128
N_COLS = 32768
K_RANK = N_COLS // 2
L = 16
NC, NS = 2, 16
NW = NC * NS
ROWS_PER_W = N_ROWS // NW
NV = N_COLS // L

B1_BITS, B2_BITS, B3_BITS = 11, 10, 10
NB1, NB2, NB3 = 1 << B1_BITS, 1 << B2_BITS, 1 << B3_BITS
SIGN_MASK = 0x7FFFFFFF
ONE_F32_BITS = 0x3F800000


def _bcast(s):
    return lax.broadcast_in_dim(s, (L,), ())


def _body(x_hbm, y_hbm, m_hbm, xrow, hist, mrow):
    c = lax.axis_index("c")
    s = lax.axis_index("s")
    wid = s * NC + c

    lane = lax.broadcasted_iota(jnp.int32, (L,), 0)
    zeros_i = jnp.zeros((L,), jnp.int32)
    ones_i = jnp.ones((L,), jnp.int32)
    onef_bits = jnp.full((L,), ONE_F32_BITS, jnp.int32)
    lane_b1 = lane * NB1
    lane_b2 = lane * NB2
    lane_b3 = lane * NB3

    def zbody(i, carry):
        hist[pl.ds(i * L, L)] = zeros_i
        return carry

    lax.fori_loop(0, (L * NB1) // L, zbody, 0)

    def magnitude(i):
        raw = xrow[pl.ds(i * L, L)]
        u = raw & SIGN_MASK
        return raw, u

    def find_bin(nbins, kprime):
        def mbody(j, carry):
            total, nless, cbefore = carry
            acc = hist[pl.ds(j * L, L)]
            hist[pl.ds(j * L, L)] = zeros_i
            for l in range(1, L):
                off = l * nbins + j * L
                acc = acc + hist[pl.ds(off, L)]
                hist[pl.ds(off, L)] = zeros_i
            cum = jnp.cumsum(acc) + _bcast(total)
            mlt = cum < _bcast(kprime)
            nless = nless + jnp.sum(jnp.where(mlt, ones_i, zeros_i))
            cbefore = jnp.maximum(cbefore, jnp.max(jnp.where(mlt, cum, zeros_i)))
            total = jnp.max(cum)
            return total, nless, cbefore

        _, nless, cbefore = lax.fori_loop(
            0, nbins // L, mbody, (jnp.int32(0), jnp.int32(0), jnp.int32(0))
        )
        return nless, cbefore

    def do_row(r, carry):
        row_base = (wid * ROWS_PER_W + r) * N_COLS
        pltpu.sync_copy(x_hbm.at[pl.ds(row_base, N_COLS)], xrow)

        def s1(i, cr):
            _, u = magnitude(i)
            b = lax.shift_right_logical(u, B2_BITS + B3_BITS)
            plsc.addupdate_scatter(hist, [lane_b1 + b], ones_i)
            return cr

        lax.fori_loop(0, NV, s1, 0)
        kprime = jnp.int32(K_RANK)
        b1, cbefore = find_bin(NB1, kprime)
        kprime = kprime - cbefore

        b1v = _bcast(b1)

        def s2(i, cr):
            _, u = magnitude(i)
            p = lax.shift_right_logical(u, B2_BITS + B3_BITS)
            b = lax.shift_right_logical(u, B3_BITS) & (NB2 - 1)
            plsc.addupdate_scatter(hist, [lane_b2 + b], ones_i, mask=p == b1v)
            return cr

        lax.fori_loop(0, NV, s2, 0)
        b2, cbefore = find_bin(NB2, kprime)
        kprime = kprime - cbefore

        prefix2 = (b1 << B2_BITS) | b2
        p2v = _bcast(prefix2)

        def s3(i, cr):
            _, u = magnitude(i)
            p = lax.shift_right_logical(u, B3_BITS)
            b = u & (NB3 - 1)
            plsc.addupdate_scatter(hist, [lane_b3 + b], ones_i, mask=p == p2v)
            return cr

        lax.fori_loop(0, NV, s3, 0)
        b3, _ = find_bin(NB3, kprime)

        thr = (prefix2 << B3_BITS) | b3
        thrv = _bcast(thr)

        def so(i, cr):
            raw, u = magnitude(i)
            keep = u >= thrv
            xrow[pl.ds(i * L, L)] = jnp.where(keep, raw, zeros_i)
            mrow[pl.ds(i * L, L)] = jnp.where(keep, onef_bits, zeros_i)
            return cr

        lax.fori_loop(0, NV, so, 0)
        pltpu.sync_copy(xrow, y_hbm.at[pl.ds(row_base, N_COLS)])
        pltpu.sync_copy(mrow, m_hbm.at[pl.ds(row_base, N_COLS)])
        return carry

    lax.fori_loop(0, ROWS_PER_W, do_row, 0)


_sparsify = functools.partial(
    pl.kernel,
    out_type=(
        jax.ShapeDtypeStruct((N_ROWS * N_COLS,), jnp.int32),
        jax.ShapeDtypeStruct((N_ROWS * N_COLS,), jnp.int32),
    ),
    mesh=plsc.VectorSubcoreMesh(
        core_axis_name="c", subcore_axis_name="s", num_cores=NC, num_subcores=NS
    ),
    scratch_types=[
        pltpu.VMEM((N_COLS,), jnp.int32),
        pltpu.VMEM((L * NB1,), jnp.int32),
        pltpu.VMEM((N_COLS,), jnp.int32),
    ],
    compiler_params=pltpu.CompilerParams(needs_layout_passes=False),
)(_body)


@jax.jit
def kernel(x):
    xi = lax.bitcast_convert_type(x.reshape(-1), jnp.int32)
    y, m = _sparsify(xi)
    y = lax.bitcast_convert_type(y, jnp.float32).reshape(x.shape)
    m = lax.bitcast_convert_type(m, jnp.float32).reshape(x.shape)
    return y, m


# speedup vs baseline: 5.2314x; 1.0569x over previous
import functools

import jax
import jax.numpy as jnp
from jax import lax
from jax.experimental import pallas as pl
from jax.experimental.pallas import tpu as pltpu
from jax.experimental.pallas import tpu_sc as plsc

N_ROWS = 128
N_COLS = 32768
K_RANK = N_COLS // 2
L = 16
NC, NS = 2, 16
NW = NC * NS
ROWS_PER_W = N_ROWS // NW
NV = N_COLS // L

B1_BITS, B2_BITS, B3_BITS = 11, 10, 10
NB1, NB2, NB3 = 1 << B1_BITS, 1 << B2_BITS, 1 << B3_BITS
SIGN_MASK = 0x7FFFFFFF
ONE_F32_BITS = 0x3F800000


def _bcast(s):
    return lax.broadcast_in_dim(s, (L,), ())


def _body(x_hbm, y_hbm, m_hbm, xrow, hist, mrow):
    c = lax.axis_index("c")
    s = lax.axis_index("s")
    wid = s * NC + c

    lane = lax.broadcasted_iota(jnp.int32, (L,), 0)
    zeros_i = jnp.zeros((L,), jnp.int32)
    ones_i = jnp.ones((L,), jnp.int32)
    onef_bits = jnp.full((L,), ONE_F32_BITS, jnp.int32)
    lane_b1 = lane * NB1
    lane_b2 = lane * NB2
    lane_b3 = lane * NB3

    def zbody(i, carry):
        hist[pl.ds(i * L, L)] = zeros_i
        return carry

    lax.fori_loop(0, (L * NB1) // L, zbody, 0, unroll=8)

    def magnitude(i):
        raw = xrow[pl.ds(i * L, L)]
        u = raw & SIGN_MASK
        return raw, u

    def find_bin(nbins, kprime):
        def mbody(j, carry):
            total, nless, cbefore = carry
            acc = hist[pl.ds(j * L, L)]
            hist[pl.ds(j * L, L)] = zeros_i
            for l in range(1, L):
                off = l * nbins + j * L
                acc = acc + hist[pl.ds(off, L)]
                hist[pl.ds(off, L)] = zeros_i
            cum = jnp.cumsum(acc) + _bcast(total)
            mlt = cum < _bcast(kprime)
            nless = nless + jnp.sum(jnp.where(mlt, ones_i, zeros_i))
            cbefore = jnp.maximum(cbefore, jnp.max(jnp.where(mlt, cum, zeros_i)))
            total = jnp.max(cum)
            return total, nless, cbefore

        _, nless, cbefore = lax.fori_loop(
            0, nbins // L, mbody, (jnp.int32(0), jnp.int32(0), jnp.int32(0)),
            unroll=2,
        )
        return nless, cbefore

    def do_row(r, carry):
        row_base = (wid * ROWS_PER_W + r) * N_COLS
        pltpu.sync_copy(x_hbm.at[pl.ds(row_base, N_COLS)], xrow)

        def s1(i, cr):
            _, u = magnitude(i)
            b = lax.shift_right_logical(u, B2_BITS + B3_BITS)
            plsc.addupdate_scatter(hist, [lane_b1 + b], ones_i)
            return cr

        lax.fori_loop(0, NV, s1, 0, unroll=8)
        kprime = jnp.int32(K_RANK)
        b1, cbefore = find_bin(NB1, kprime)
        kprime = kprime - cbefore

        b1v = _bcast(b1)

        def s2(i, cr):
            _, u = magnitude(i)
            p = lax.shift_right_logical(u, B2_BITS + B3_BITS)
            b = lax.shift_right_logical(u, B3_BITS) & (NB2 - 1)
            plsc.addupdate_scatter(hist, [lane_b2 + b], ones_i, mask=p == b1v)
            return cr

        lax.fori_loop(0, NV, s2, 0, unroll=8)
        b2, cbefore = find_bin(NB2, kprime)
        kprime = kprime - cbefore

        prefix2 = (b1 << B2_BITS) | b2
        p2v = _bcast(prefix2)

        def s3(i, cr):
            _, u = magnitude(i)
            p = lax.shift_right_logical(u, B3_BITS)
            b = u & (NB3 - 1)
            plsc.addupdate_scatter(hist, [lane_b3 + b], ones_i, mask=p == p2v)
            return cr

        lax.fori_loop(0, NV, s3, 0, unroll=8)
        b3, _ = find_bin(NB3, kprime)

        thr = (prefix2 << B3_BITS) | b3
        thrv = _bcast(thr)

        def so(i, cr):
            raw, u = magnitude(i)
            keep = u >= thrv
            xrow[pl.ds(i * L, L)] = jnp.where(keep, raw, zeros_i)
            mrow[pl.ds(i * L, L)] = jnp.where(keep, onef_bits, zeros_i)
            return cr

        lax.fori_loop(0, NV, so, 0, unroll=8)
        pltpu.sync_copy(xrow, y_hbm.at[pl.ds(row_base, N_COLS)])
        pltpu.sync_copy(mrow, m_hbm.at[pl.ds(row_base, N_COLS)])
        return carry

    lax.fori_loop(0, ROWS_PER_W, do_row, 0)


_sparsify = functools.partial(
    pl.kernel,
    out_type=(
        jax.ShapeDtypeStruct((N_ROWS * N_COLS,), jnp.int32),
        jax.ShapeDtypeStruct((N_ROWS * N_COLS,), jnp.int32),
    ),
    mesh=plsc.VectorSubcoreMesh(
        core_axis_name="c", subcore_axis_name="s", num_cores=NC, num_subcores=NS
    ),
    scratch_types=[
        pltpu.VMEM((N_COLS,), jnp.int32),
        pltpu.VMEM((L * NB1,), jnp.int32),
        pltpu.VMEM((N_COLS,), jnp.int32),
    ],
    compiler_params=pltpu.CompilerParams(needs_layout_passes=False),
)(_body)


@jax.jit
def kernel(x):
    xi = lax.bitcast_convert_type(x.reshape(-1), jnp.int32)
    y, m = _sparsify(xi)
    y = lax.bitcast_convert_type(y, jnp.float32).reshape(x.shape)
    m = lax.bitcast_convert_type(m, jnp.float32).reshape(x.shape)
    return y, m


# speedup vs baseline: 11.1007x; 2.1219x over previous
import functools

import jax
import jax.numpy as jnp
from jax import lax
from jax.experimental import pallas as pl
from jax.experimental.pallas import tpu as pltpu
from jax.experimental.pallas import tpu_sc as plsc

N_ROWS = 128
N_COLS = 32768
K_RANK = N_COLS // 2
L = 16
NC, NS = 2, 16
NW = NC * NS
ROWS_PER_W = N_ROWS // NW
NV = N_COLS // L

B1_BITS, B2_BITS, B3_BITS = 11, 10, 10
NB1, NB2, NB3 = 1 << B1_BITS, 1 << B2_BITS, 1 << B3_BITS
SIGN_MASK = 0x7FFFFFFF
ONE_F32_BITS = 0x3F800000


def _bcast(s):
    return lax.broadcast_in_dim(s, (L,), ())


def _body(x_hbm, y_hbm, m_hbm, xrow, hist, mrow):
    c = lax.axis_index("c")
    s = lax.axis_index("s")
    wid = s * NC + c

    lane = lax.broadcasted_iota(jnp.int32, (L,), 0)
    zeros_i = jnp.zeros((L,), jnp.int32)
    ones_i = jnp.ones((L,), jnp.int32)
    onef_bits = jnp.full((L,), ONE_F32_BITS, jnp.int32)
    lane_b1 = lane * NB1
    lane_b2 = lane * NB2
    lane_b3 = lane * NB3

    @plsc.parallel_loop(0, (L * NB1) // L, unroll=8)
    def zbody(i):
        hist[pl.ds(i * L, L)] = zeros_i

    def magnitude(i):
        raw = xrow[pl.ds(i * L, L)]
        u = raw & SIGN_MASK
        return raw, u

    def find_bin(nbins, kprime):
        @plsc.parallel_loop(
            0, nbins // L, unroll=2,
            carry=(jnp.int32(0), jnp.int32(0), jnp.int32(0)),
        )
        def mcarry(j, carry):
            total, nless, cbefore = carry
            acc = hist[pl.ds(j * L, L)]
            hist[pl.ds(j * L, L)] = zeros_i
            for l in range(1, L):
                off = l * nbins + j * L
                acc = acc + hist[pl.ds(off, L)]
                hist[pl.ds(off, L)] = zeros_i
            cum = jnp.cumsum(acc) + _bcast(total)
            mlt = cum < _bcast(kprime)
            nless = nless + jnp.sum(jnp.where(mlt, ones_i, zeros_i))
            cbefore = jnp.maximum(cbefore, jnp.max(jnp.where(mlt, cum, zeros_i)))
            total = jnp.max(cum)
            return total, nless, cbefore

        _, nless, cbefore = mcarry
        return nless, cbefore

    def do_row(r, carry):
        row_base = (wid * ROWS_PER_W + r) * N_COLS
        pltpu.sync_copy(x_hbm.at[pl.ds(row_base, N_COLS)], xrow)

        @plsc.parallel_loop(0, NV, unroll=8)
        def s1(i):
            _, u = magnitude(i)
            b = lax.shift_right_logical(u, B2_BITS + B3_BITS)
            plsc.addupdate_scatter(hist, [lane_b1 + b], ones_i)
        kprime = jnp.int32(K_RANK)
        b1, cbefore = find_bin(NB1, kprime)
        kprime = kprime - cbefore

        b1v = _bcast(b1)

        @plsc.parallel_loop(0, NV, unroll=8)
        def s2(i):
            _, u = magnitude(i)
            p = lax.shift_right_logical(u, B2_BITS + B3_BITS)
            b = lax.shift_right_logical(u, B3_BITS) & (NB2 - 1)
            plsc.addupdate_scatter(hist, [lane_b2 + b], ones_i, mask=p == b1v)
        b2, cbefore = find_bin(NB2, kprime)
        kprime = kprime - cbefore

        prefix2 = (b1 << B2_BITS) | b2
        p2v = _bcast(prefix2)

        @plsc.parallel_loop(0, NV, unroll=8)
        def s3(i):
            _, u = magnitude(i)
            p = lax.shift_right_logical(u, B3_BITS)
            b = u & (NB3 - 1)
            plsc.addupdate_scatter(hist, [lane_b3 + b], ones_i, mask=p == p2v)
        b3, _ = find_bin(NB3, kprime)

        thr = (prefix2 << B3_BITS) | b3
        thrv = _bcast(thr)

        @plsc.parallel_loop(0, NV, unroll=8)
        def so(i):
            raw, u = magnitude(i)
            keep = u >= thrv
            xrow[pl.ds(i * L, L)] = jnp.where(keep, raw, zeros_i)
            mrow[pl.ds(i * L, L)] = jnp.where(keep, onef_bits, zeros_i)
        pltpu.sync_copy(xrow, y_hbm.at[pl.ds(row_base, N_COLS)])
        pltpu.sync_copy(mrow, m_hbm.at[pl.ds(row_base, N_COLS)])
        return carry

    lax.fori_loop(0, ROWS_PER_W, do_row, 0)


_sparsify = functools.partial(
    pl.kernel,
    out_type=(
        jax.ShapeDtypeStruct((N_ROWS * N_COLS,), jnp.int32),
        jax.ShapeDtypeStruct((N_ROWS * N_COLS,), jnp.int32),
    ),
    mesh=plsc.VectorSubcoreMesh(
        core_axis_name="c", subcore_axis_name="s", num_cores=NC, num_subcores=NS
    ),
    scratch_types=[
        pltpu.VMEM((N_COLS,), jnp.int32),
        pltpu.VMEM((L * NB1,), jnp.int32),
        pltpu.VMEM((N_COLS,), jnp.int32),
    ],
    compiler_params=pltpu.CompilerParams(needs_layout_passes=False),
)(_body)


@jax.jit
def kernel(x):
    xi = lax.bitcast_convert_type(x.reshape(-1), jnp.int32)
    y, m = _sparsify(xi)
    y = lax.bitcast_convert_type(y, jnp.float32).reshape(x.shape)
    m = lax.bitcast_convert_type(m, jnp.float32).reshape(x.shape)
    return y, m


# speedup vs baseline: 11.1938x; 1.0084x over previous
import functools

import jax
import jax.numpy as jnp
from jax import lax
from jax.experimental import pallas as pl
from jax.experimental.pallas import tpu as pltpu
from jax.experimental.pallas import tpu_sc as plsc

N_ROWS = 128
N_COLS = 32768
K_RANK = N_COLS // 2
L = 16
NC, NS = 2, 16
NW = NC * NS
ROWS_PER_W = N_ROWS // NW
NV = N_COLS // L

B1_BITS, B2_BITS, B3_BITS = 11, 10, 10
NB1, NB2, NB3 = 1 << B1_BITS, 1 << B2_BITS, 1 << B3_BITS
SIGN_MASK = 0x7FFFFFFF
ONE_F32_BITS = 0x3F800000


def _bcast(s):
    return lax.broadcast_in_dim(s, (L,), ())


def _body(x_hbm, y_hbm, m_hbm, xrow, hist, mrow):
    c = lax.axis_index("c")
    s = lax.axis_index("s")
    wid = s * NC + c

    zeros_i = jnp.zeros((L,), jnp.int32)
    ones_i = jnp.ones((L,), jnp.int32)
    onef_bits = jnp.full((L,), ONE_F32_BITS, jnp.int32)

    @plsc.parallel_loop(0, NB1 // L, unroll=8)
    def zbody(i):
        hist[pl.ds(i * L, L)] = zeros_i

    def magnitude(i):
        raw = xrow[pl.ds(i * L, L)]
        u = raw & SIGN_MASK
        return raw, u

    def find_bin(nbins, kprime):
        @plsc.parallel_loop(
            0, nbins // L, unroll=2,
            carry=(jnp.int32(0), jnp.int32(0), jnp.int32(0)),
        )
        def mcarry(j, carry):
            total, nless, cbefore = carry
            acc = hist[pl.ds(j * L, L)]
            hist[pl.ds(j * L, L)] = zeros_i
            cum = jnp.cumsum(acc) + _bcast(total)
            mlt = cum < _bcast(kprime)
            nless = nless + jnp.sum(jnp.where(mlt, ones_i, zeros_i))
            cbefore = jnp.maximum(cbefore, jnp.max(jnp.where(mlt, cum, zeros_i)))
            total = jnp.max(cum)
            return total, nless, cbefore

        _, nless, cbefore = mcarry
        return nless, cbefore

    def do_row(r, carry):
        row_base = (wid * ROWS_PER_W + r) * N_COLS
        pltpu.sync_copy(x_hbm.at[pl.ds(row_base, N_COLS)], xrow)

        @plsc.parallel_loop(0, NV, unroll=8)
        def s1(i):
            _, u = magnitude(i)
            b = lax.shift_right_logical(u, B2_BITS + B3_BITS)
            plsc.addupdate_scatter(hist, [b], ones_i)
        kprime = jnp.int32(K_RANK)
        b1, cbefore = find_bin(NB1, kprime)
        kprime = kprime - cbefore

        b1v = _bcast(b1)

        @plsc.parallel_loop(0, NV, unroll=8)
        def s2(i):
            _, u = magnitude(i)
            p = lax.shift_right_logical(u, B2_BITS + B3_BITS)
            b = lax.shift_right_logical(u, B3_BITS) & (NB2 - 1)
            plsc.addupdate_scatter(hist, [b], ones_i, mask=p == b1v)
        b2, cbefore = find_bin(NB2, kprime)
        kprime = kprime - cbefore

        prefix2 = (b1 << B2_BITS) | b2
        p2v = _bcast(prefix2)

        @plsc.parallel_loop(0, NV, unroll=8)
        def s3(i):
            _, u = magnitude(i)
            p = lax.shift_right_logical(u, B3_BITS)
            b = u & (NB3 - 1)
            plsc.addupdate_scatter(hist, [b], ones_i, mask=p == p2v)
        b3, _ = find_bin(NB3, kprime)

        thr = (prefix2 << B3_BITS) | b3
        thrv = _bcast(thr)

        @plsc.parallel_loop(0, NV, unroll=8)
        def so(i):
            raw, u = magnitude(i)
            keep = u >= thrv
            xrow[pl.ds(i * L, L)] = jnp.where(keep, raw, zeros_i)
            mrow[pl.ds(i * L, L)] = jnp.where(keep, onef_bits, zeros_i)
        pltpu.sync_copy(xrow, y_hbm.at[pl.ds(row_base, N_COLS)])
        pltpu.sync_copy(mrow, m_hbm.at[pl.ds(row_base, N_COLS)])
        return carry

    lax.fori_loop(0, ROWS_PER_W, do_row, 0)


_sparsify = functools.partial(
    pl.kernel,
    out_type=(
        jax.ShapeDtypeStruct((N_ROWS * N_COLS,), jnp.int32),
        jax.ShapeDtypeStruct((N_ROWS * N_COLS,), jnp.int32),
    ),
    mesh=plsc.VectorSubcoreMesh(
        core_axis_name="c", subcore_axis_name="s", num_cores=NC, num_subcores=NS
    ),
    scratch_types=[
        pltpu.VMEM((N_COLS,), jnp.int32),
        pltpu.VMEM((NB1,), jnp.int32),
        pltpu.VMEM((N_COLS,), jnp.int32),
    ],
    compiler_params=pltpu.CompilerParams(needs_layout_passes=False),
)(_body)


@jax.jit
def kernel(x):
    xi = lax.bitcast_convert_type(x.reshape(-1), jnp.int32)
    y, m = _sparsify(xi)
    y = lax.bitcast_convert_type(y, jnp.float32).reshape(x.shape)
    m = lax.bitcast_convert_type(m, jnp.float32).reshape(x.shape)
    return y, m


# speedup vs baseline: 11.3120x; 1.0106x over previous
import functools

import jax
import jax.numpy as jnp
from jax import lax
from jax.experimental import pallas as pl
from jax.experimental.pallas import tpu as pltpu
from jax.experimental.pallas import tpu_sc as plsc

N_ROWS = 128
N_COLS = 32768
K_RANK = N_COLS // 2
L = 16
NC, NS = 2, 16
NW = NC * NS
ROWS_PER_W = N_ROWS // NW
NV = N_COLS // L

B1_BITS, B2_BITS, B3_BITS = 11, 10, 10
NB1, NB2, NB3 = 1 << B1_BITS, 1 << B2_BITS, 1 << B3_BITS
SIGN_MASK = 0x7FFFFFFF


def _bcast(s):
    return lax.broadcast_in_dim(s, (L,), ())


def _thr_body(x_hbm, thr_hbm, xrow, hist, tbuf):
    c = lax.axis_index("c")
    s = lax.axis_index("s")
    wid = s * NC + c

    lane = lax.broadcasted_iota(jnp.int32, (L,), 0)
    zeros_i = jnp.zeros((L,), jnp.int32)
    ones_i = jnp.ones((L,), jnp.int32)

    @plsc.parallel_loop(0, NB1 // L, unroll=8)
    def zbody(i):
        hist[pl.ds(i * L, L)] = zeros_i

    def magnitude(i):
        raw = xrow[pl.ds(i * L, L)]
        u = raw & SIGN_MASK
        return raw, u

    def find_bin(nbins, kprime):
        @plsc.parallel_loop(
            0, nbins // L, unroll=2,
            carry=(jnp.int32(0), jnp.int32(0), jnp.int32(0)),
        )
        def mcarry(j, carry):
            total, nless, cbefore = carry
            acc = hist[pl.ds(j * L, L)]
            hist[pl.ds(j * L, L)] = zeros_i
            cum = jnp.cumsum(acc) + _bcast(total)
            mlt = cum < _bcast(kprime)
            nless = nless + jnp.sum(jnp.where(mlt, ones_i, zeros_i))
            cbefore = jnp.maximum(cbefore, jnp.max(jnp.where(mlt, cum, zeros_i)))
            total = jnp.max(cum)
            return total, nless, cbefore

        _, nless, cbefore = mcarry
        return nless, cbefore

    def do_row(r, thrvec):
        row_base = (wid * ROWS_PER_W + r) * N_COLS
        pltpu.sync_copy(x_hbm.at[pl.ds(row_base, N_COLS)], xrow)

        @plsc.parallel_loop(0, NV, unroll=8)
        def s1(i):
            _, u = magnitude(i)
            b = lax.shift_right_logical(u, B2_BITS + B3_BITS)
            plsc.addupdate_scatter(hist, [b], ones_i)

        kprime = jnp.int32(K_RANK)
        b1, cbefore = find_bin(NB1, kprime)
        kprime = kprime - cbefore

        b1v = _bcast(b1)

        @plsc.parallel_loop(0, NV, unroll=8)
        def s2(i):
            _, u = magnitude(i)
            p = lax.shift_right_logical(u, B2_BITS + B3_BITS)
            b = lax.shift_right_logical(u, B3_BITS) & (NB2 - 1)
            plsc.addupdate_scatter(hist, [b], ones_i, mask=p == b1v)

        b2, cbefore = find_bin(NB2, kprime)
        kprime = kprime - cbefore

        prefix2 = (b1 << B2_BITS) | b2
        p2v = _bcast(prefix2)

        @plsc.parallel_loop(0, NV, unroll=8)
        def s3(i):
            _, u = magnitude(i)
            p = lax.shift_right_logical(u, B3_BITS)
            b = u & (NB3 - 1)
            plsc.addupdate_scatter(hist, [b], ones_i, mask=p == p2v)

        b3, _ = find_bin(NB3, kprime)

        thr = (prefix2 << B3_BITS) | b3
        return jnp.where(lane == _bcast(r), _bcast(thr), thrvec)

    thrvec = lax.fori_loop(0, ROWS_PER_W, do_row, zeros_i)
    tbuf[...] = thrvec
    pltpu.sync_copy(tbuf, thr_hbm.at[pl.ds(wid * L, L)])


_sc_thresholds = functools.partial(
    pl.kernel,
    out_type=jax.ShapeDtypeStruct((NW * L,), jnp.int32),
    mesh=plsc.VectorSubcoreMesh(
        core_axis_name="c", subcore_axis_name="s", num_cores=NC, num_subcores=NS
    ),
    scratch_types=[
        pltpu.VMEM((N_COLS,), jnp.int32),
        pltpu.VMEM((NB1,), jnp.int32),
        pltpu.VMEM((L,), jnp.int32),
    ],
    compiler_params=pltpu.CompilerParams(needs_layout_passes=False),
)(_thr_body)


BR, BC = 8, 4096


def _mask_body(thr_ref, x_ref, y_ref, m_ref):
    xb = x_ref[...]
    keep = jnp.abs(xb) >= thr_ref[...]
    y_ref[...] = jnp.where(keep, xb, 0.0)
    m_ref[...] = keep.astype(jnp.float32)


_apply_mask = pl.pallas_call(
    _mask_body,
    grid=(N_ROWS // BR, N_COLS // BC),
    in_specs=[
        pl.BlockSpec((BR, 1), lambda i, j: (i, 0)),
        pl.BlockSpec((BR, BC), lambda i, j: (i, j)),
    ],
    out_specs=[
        pl.BlockSpec((BR, BC), lambda i, j: (i, j)),
        pl.BlockSpec((BR, BC), lambda i, j: (i, j)),
    ],
    out_shape=[
        jax.ShapeDtypeStruct((N_ROWS, N_COLS), jnp.float32),
        jax.ShapeDtypeStruct((N_ROWS, N_COLS), jnp.float32),
    ],
)


@jax.jit
def kernel(x):
    xi = lax.bitcast_convert_type(x.reshape(-1), jnp.int32)
    thr_flat = _sc_thresholds(xi)
    thr_bits = thr_flat.reshape(NW, L)[:, :ROWS_PER_W].reshape(N_ROWS, 1)
    thr = lax.bitcast_convert_type(thr_bits, jnp.float32)
    y, m = _apply_mask(thr, x)
    return y, m


# speedup vs baseline: 11.3182x; 1.0006x over previous
import functools

import jax
import jax.numpy as jnp
from jax import lax
from jax.experimental import pallas as pl
from jax.experimental.pallas import tpu as pltpu
from jax.experimental.pallas import tpu_sc as plsc

N_ROWS = 128
N_COLS = 32768
K_RANK = N_COLS // 2
L = 16
NC, NS = 2, 16
NW = NC * NS
ROWS_PER_W = N_ROWS // NW
NV = N_COLS // L

B1_BITS, B2_BITS, B3_BITS = 11, 10, 10
NB1, NB2, NB3 = 1 << B1_BITS, 1 << B2_BITS, 1 << B3_BITS
SPLIT = 8
SIGN_MASK = 0x7FFFFFFF


def _bcast(s):
    return lax.broadcast_in_dim(s, (L,), ())


def _thr_body(x_hbm, thr_hbm, xrow, hist, tbuf):
    c = lax.axis_index("c")
    s = lax.axis_index("s")
    wid = s * NC + c

    lane = lax.broadcasted_iota(jnp.int32, (L,), 0)
    zeros_i = jnp.zeros((L,), jnp.int32)
    ones_i = jnp.ones((L,), jnp.int32)

    @plsc.parallel_loop(0, SPLIT * NB1 // L, unroll=8)
    def zbody(i):
        hist[pl.ds(i * L, L)] = zeros_i

    def magnitude(i):
        raw = xrow[pl.ds(i * L, L)]
        u = raw & SIGN_MASK
        return raw, u

    def find_bin(nbins, kprime):
        @plsc.parallel_loop(
            0, nbins // L, unroll=2,
            carry=(jnp.int32(0), jnp.int32(0), jnp.int32(0)),
        )
        def mcarry(j, carry):
            total, nless, cbefore = carry
            acc = hist[pl.ds(j * L, L)]
            hist[pl.ds(j * L, L)] = zeros_i
            for sub in range(1, SPLIT):
                off = sub * nbins + j * L
                acc = acc + hist[pl.ds(off, L)]
                hist[pl.ds(off, L)] = zeros_i
            cum = jnp.cumsum(acc) + _bcast(total)
            mlt = cum < _bcast(kprime)
            nless = nless + jnp.sum(jnp.where(mlt, ones_i, zeros_i))
            cbefore = jnp.maximum(cbefore, jnp.max(jnp.where(mlt, cum, zeros_i)))
            total = jnp.max(cum)
            return total, nless, cbefore

        _, nless, cbefore = mcarry
        return nless, cbefore

    def do_row(r, thrvec):
        row_base = (wid * ROWS_PER_W + r) * N_COLS
        pltpu.sync_copy(x_hbm.at[pl.ds(row_base, N_COLS)], xrow)

        @plsc.parallel_loop(0, NV, unroll=8)
        def s1(i):
            _, u = magnitude(i)
            b = lax.shift_right_logical(u, B2_BITS + B3_BITS)
            plsc.addupdate_scatter(hist, [(i & (SPLIT - 1)) * NB1 + b], ones_i)

        kprime = jnp.int32(K_RANK)
        b1, cbefore = find_bin(NB1, kprime)
        kprime = kprime - cbefore

        b1v = _bcast(b1)

        @plsc.parallel_loop(0, NV, unroll=8)
        def s2(i):
            _, u = magnitude(i)
            p = lax.shift_right_logical(u, B2_BITS + B3_BITS)
            b = lax.shift_right_logical(u, B3_BITS) & (NB2 - 1)
            plsc.addupdate_scatter(
                hist, [(i & (SPLIT - 1)) * NB2 + b], ones_i, mask=p == b1v)

        b2, cbefore = find_bin(NB2, kprime)
        kprime = kprime - cbefore

        prefix2 = (b1 << B2_BITS) | b2
        p2v = _bcast(prefix2)

        @plsc.parallel_loop(0, NV, unroll=8)
        def s3(i):
            _, u = magnitude(i)
            p = lax.shift_right_logical(u, B3_BITS)
            b = u & (NB3 - 1)
            plsc.addupdate_scatter(
                hist, [(i & (SPLIT - 1)) * NB3 + b], ones_i, mask=p == p2v)

        b3, _ = find_bin(NB3, kprime)

        thr = (prefix2 << B3_BITS) | b3
        return jnp.where(lane == _bcast(r), _bcast(thr), thrvec)

    thrvec = lax.fori_loop(0, ROWS_PER_W, do_row, zeros_i)
    tbuf[...] = thrvec
    pltpu.sync_copy(tbuf, thr_hbm.at[pl.ds(wid * L, L)])


_sc_thresholds = functools.partial(
    pl.kernel,
    out_type=jax.ShapeDtypeStruct((NW * L,), jnp.int32),
    mesh=plsc.VectorSubcoreMesh(
        core_axis_name="c", subcore_axis_name="s", num_cores=NC, num_subcores=NS
    ),
    scratch_types=[
        pltpu.VMEM((N_COLS,), jnp.int32),
        pltpu.VMEM((SPLIT * NB1,), jnp.int32),
        pltpu.VMEM((L,), jnp.int32),
    ],
    compiler_params=pltpu.CompilerParams(needs_layout_passes=False),
)(_thr_body)


BR, BC = 8, 4096


def _mask_body(thr_ref, x_ref, y_ref, m_ref):
    xb = x_ref[...]
    keep = jnp.abs(xb) >= thr_ref[...]
    y_ref[...] = jnp.where(keep, xb, 0.0)
    m_ref[...] = keep.astype(jnp.float32)


_apply_mask = pl.pallas_call(
    _mask_body,
    grid=(N_ROWS // BR, N_COLS // BC),
    in_specs=[
        pl.BlockSpec((BR, 1), lambda i, j: (i, 0)),
        pl.BlockSpec((BR, BC), lambda i, j: (i, j)),
    ],
    out_specs=[
        pl.BlockSpec((BR, BC), lambda i, j: (i, j)),
        pl.BlockSpec((BR, BC), lambda i, j: (i, j)),
    ],
    out_shape=[
        jax.ShapeDtypeStruct((N_ROWS, N_COLS), jnp.float32),
        jax.ShapeDtypeStruct((N_ROWS, N_COLS), jnp.float32),
    ],
)


@jax.jit
def kernel(x):
    xi = lax.bitcast_convert_type(x.reshape(-1), jnp.int32)
    thr_flat = _sc_thresholds(xi)
    thr_bits = thr_flat.reshape(NW, L)[:, :ROWS_PER_W].reshape(N_ROWS, 1)
    thr = lax.bitcast_convert_type(thr_bits, jnp.float32)
    y, m = _apply_mask(thr, x)
    return y, m


# speedup vs baseline: 15.3475x; 1.3560x over previous
import functools

import jax
import jax.numpy as jnp
from jax import lax
from jax.experimental import pallas as pl
from jax.experimental.pallas import tpu as pltpu
from jax.experimental.pallas import tpu_sc as plsc

N_ROWS = 128
N_COLS = 32768
K_RANK = N_COLS // 2
L = 16
NC, NS = 2, 16
NW = NC * NS
ROWS_PER_W = N_ROWS // NW
NV = N_COLS // L

B1_BITS, B2_BITS, B3_BITS = 11, 10, 10
NB1, NB2, NB3 = 1 << B1_BITS, 1 << B2_BITS, 1 << B3_BITS
SPLIT = 8
SIGN_MASK = 0x7FFFFFFF


def _bcast(s):
    return lax.broadcast_in_dim(s, (L,), ())


def _thr_body(x_hbm, thr_hbm, xrow, hist, tbuf):
    c = lax.axis_index("c")
    s = lax.axis_index("s")
    wid = s * NC + c

    lane = lax.broadcasted_iota(jnp.int32, (L,), 0)
    zeros_i = jnp.zeros((L,), jnp.int32)
    ones_i = jnp.ones((L,), jnp.int32)

    @plsc.parallel_loop(0, SPLIT * NB1 // L, unroll=8)
    def zbody(i):
        hist[pl.ds(i * L, L)] = zeros_i

    def magnitude(i):
        raw = xrow[pl.ds(i * L, L)]
        u = raw & SIGN_MASK
        return raw, u

    def find_bin(nbins, kprime):
        @plsc.parallel_loop(
            0, nbins // L, unroll=2,
            carry=(jnp.int32(0), jnp.int32(0), jnp.int32(0)),
        )
        def mcarry(j, carry):
            total, nless, cbefore = carry
            acc = hist[pl.ds(j * L, L)]
            hist[pl.ds(j * L, L)] = zeros_i
            for sub in range(1, SPLIT):
                off = sub * nbins + j * L
                acc = acc + hist[pl.ds(off, L)]
                hist[pl.ds(off, L)] = zeros_i
            cum = jnp.cumsum(acc) + _bcast(total)
            mlt = cum < _bcast(kprime)
            nless = nless + jnp.sum(jnp.where(mlt, ones_i, zeros_i))
            cbefore = jnp.maximum(cbefore, jnp.max(jnp.where(mlt, cum, zeros_i)))
            total = jnp.max(cum)
            return total, nless, cbefore

        _, nless, cbefore = mcarry
        return nless, cbefore

    def do_row(r, thrvec):
        row_base = (wid * ROWS_PER_W + r) * N_COLS
        pltpu.sync_copy(x_hbm.at[pl.ds(row_base, N_COLS)], xrow)

        @plsc.parallel_loop(0, NV, unroll=8)
        def s1(i):
            _, u = magnitude(i)
            b = lax.shift_right_logical(u, B2_BITS + B3_BITS)
            plsc.addupdate_scatter(hist, [(i & (SPLIT - 1)) * NB1 + b], ones_i)

        kprime = jnp.int32(K_RANK)
        b1, cbefore = find_bin(NB1, kprime)
        kprime = kprime - cbefore

        b1v = _bcast(b1)

        @plsc.parallel_loop(0, NV, unroll=8)
        def s2(i):
            _, u = magnitude(i)
            p = lax.shift_right_logical(u, B2_BITS + B3_BITS)
            b = lax.shift_right_logical(u, B3_BITS) & (NB2 - 1)
            plsc.addupdate_scatter(
                hist, [(i & (SPLIT - 1)) * NB2 + b], ones_i, mask=p == b1v)

        b2, cbefore = find_bin(NB2, kprime)
        kprime = kprime - cbefore

        prefix2 = (b1 << B2_BITS) | b2
        p2v = _bcast(prefix2)

        @plsc.parallel_loop(0, NV, unroll=8)
        def s3(i):
            _, u = magnitude(i)
            p = lax.shift_right_logical(u, B3_BITS)
            b = u & (NB3 - 1)
            plsc.addupdate_scatter(
                hist, [(i & (SPLIT - 1)) * NB3 + b], ones_i, mask=p == p2v)

        b3, _ = find_bin(NB3, kprime)

        thr = (prefix2 << B3_BITS) | b3
        return jnp.where(lane == _bcast(r), _bcast(thr), thrvec)

    thrvec = lax.fori_loop(0, ROWS_PER_W, do_row, zeros_i)
    tbuf[...] = thrvec
    pltpu.sync_copy(tbuf, thr_hbm.at[pl.ds(wid * L, L)])


_sc_thresholds = functools.partial(
    pl.kernel,
    out_type=jax.ShapeDtypeStruct((NW * L,), jnp.int32),
    mesh=plsc.VectorSubcoreMesh(
        core_axis_name="c", subcore_axis_name="s", num_cores=NC, num_subcores=NS
    ),
    scratch_types=[
        pltpu.VMEM((N_COLS,), jnp.int32),
        pltpu.VMEM((SPLIT * NB1,), jnp.int32),
        pltpu.VMEM((L,), jnp.int32),
    ],
    compiler_params=pltpu.CompilerParams(needs_layout_passes=False),
)(_thr_body)


BR, BC = 8, 16384


def _mask_body(thr_ref, x_ref, y_ref, m_ref):
    xb = x_ref[...]
    keep = jnp.abs(xb) >= thr_ref[...]
    y_ref[...] = jnp.where(keep, xb, 0.0)
    m_ref[...] = keep.astype(jnp.float32)


_apply_mask = pl.pallas_call(
    _mask_body,
    grid=(N_ROWS // BR, N_COLS // BC),
    in_specs=[
        pl.BlockSpec((BR, 1), lambda i, j: (i, 0)),
        pl.BlockSpec((BR, BC), lambda i, j: (i, j)),
    ],
    out_specs=[
        pl.BlockSpec((BR, BC), lambda i, j: (i, j)),
        pl.BlockSpec((BR, BC), lambda i, j: (i, j)),
    ],
    out_shape=[
        jax.ShapeDtypeStruct((N_ROWS, N_COLS), jnp.float32),
        jax.ShapeDtypeStruct((N_ROWS, N_COLS), jnp.float32),
    ],
)


@jax.jit
def kernel(x):
    xi = lax.bitcast_convert_type(x.reshape(-1), jnp.int32)
    thr_flat = _sc_thresholds(xi)
    thr_bits = thr_flat.reshape(NW, L)[:, :ROWS_PER_W].reshape(N_ROWS, 1)
    thr = lax.bitcast_convert_type(thr_bits, jnp.float32)
    y, m = _apply_mask(thr, x)
    return y, m


# speedup vs baseline: 16.4144x; 1.0695x over previous
import functools

import jax
import jax.numpy as jnp
from jax import lax
from jax.experimental import pallas as pl
from jax.experimental.pallas import tpu as pltpu
from jax.experimental.pallas import tpu_sc as plsc

N_ROWS = 128
N_COLS = 32768
K_RANK = N_COLS // 2
L = 16
NC, NS = 2, 16
NW = NC * NS
ROWS_PER_W = N_ROWS // NW
NV = N_COLS // L

B1_BITS, B2_BITS, B3_BITS = 11, 10, 10
NB1, NB2, NB3 = 1 << B1_BITS, 1 << B2_BITS, 1 << B3_BITS
SPLIT = 8
SIGN_MASK = 0x7FFFFFFF


def _bcast(s):
    return lax.broadcast_in_dim(s, (L,), ())


def _thr_body(x_hbm, thr_hbm, xrow, hist, tbuf):
    c = lax.axis_index("c")
    s = lax.axis_index("s")
    wid = s * NC + c

    lane = lax.broadcasted_iota(jnp.int32, (L,), 0)
    zeros_i = jnp.zeros((L,), jnp.int32)
    ones_i = jnp.ones((L,), jnp.int32)

    @plsc.parallel_loop(0, SPLIT * NB1 // L, unroll=8)
    def zbody(i):
        hist[pl.ds(i * L, L)] = zeros_i

    def magnitude(i):
        raw = xrow[pl.ds(i * L, L)]
        u = raw & SIGN_MASK
        return raw, u

    def find_bin(nbins, kprime):
        @plsc.parallel_loop(
            0, nbins // L, unroll=2,
            carry=(jnp.int32(0), jnp.int32(0), jnp.int32(0)),
        )
        def mcarry(j, carry):
            total, nless, cbefore = carry
            acc = hist[pl.ds(j * L, L)]
            hist[pl.ds(j * L, L)] = zeros_i
            for sub in range(1, SPLIT):
                off = sub * nbins + j * L
                acc = acc + hist[pl.ds(off, L)]
                hist[pl.ds(off, L)] = zeros_i
            cum = jnp.cumsum(acc) + _bcast(total)
            mlt = cum < _bcast(kprime)
            nless = nless + jnp.sum(jnp.where(mlt, ones_i, zeros_i))
            cbefore = jnp.maximum(cbefore, jnp.max(jnp.where(mlt, cum, zeros_i)))
            total = jnp.max(cum)
            return total, nless, cbefore

        _, nless, cbefore = mcarry
        return nless, cbefore

    def do_row(r, thrvec):
        row_base = (wid * ROWS_PER_W + r) * N_COLS
        pltpu.sync_copy(x_hbm.at[pl.ds(row_base, N_COLS)], xrow)

        @plsc.parallel_loop(0, NV, unroll=8)
        def s1(i):
            _, u = magnitude(i)
            b = lax.shift_right_logical(u, B2_BITS + B3_BITS)
            plsc.addupdate_scatter(hist, [(i & (SPLIT - 1)) * NB1 + b], ones_i)

        kprime = jnp.int32(K_RANK)
        b1, cbefore = find_bin(NB1, kprime)
        kprime = kprime - cbefore

        b1v = _bcast(b1)

        @plsc.parallel_loop(0, NV, unroll=8)
        def s2(i):
            _, u = magnitude(i)
            p = lax.shift_right_logical(u, B2_BITS + B3_BITS)
            b = lax.shift_right_logical(u, B3_BITS) & (NB2 - 1)
            plsc.addupdate_scatter(
                hist, [(i & (SPLIT - 1)) * NB2 + b], ones_i, mask=p == b1v)

        b2, cbefore = find_bin(NB2, kprime)
        kprime = kprime - cbefore

        prefix2 = (b1 << B2_BITS) | b2
        p2v = _bcast(prefix2)

        @plsc.parallel_loop(0, NV, unroll=8)
        def s3(i):
            _, u = magnitude(i)
            p = lax.shift_right_logical(u, B3_BITS)
            b = u & (NB3 - 1)
            plsc.addupdate_scatter(
                hist, [(i & (SPLIT - 1)) * NB3 + b], ones_i, mask=p == p2v)

        b3, _ = find_bin(NB3, kprime)

        thr = (prefix2 << B3_BITS) | b3
        return jnp.where(lane == _bcast(r), _bcast(thr), thrvec)

    thrvec = lax.fori_loop(0, ROWS_PER_W, do_row, zeros_i)
    tbuf[...] = thrvec
    pltpu.sync_copy(tbuf, thr_hbm.at[pl.ds(wid * L, L)])


_sc_thresholds = functools.partial(
    pl.kernel,
    out_type=jax.ShapeDtypeStruct((NW * L,), jnp.int32),
    mesh=plsc.VectorSubcoreMesh(
        core_axis_name="c", subcore_axis_name="s", num_cores=NC, num_subcores=NS
    ),
    scratch_types=[
        pltpu.VMEM((N_COLS,), jnp.int32),
        pltpu.VMEM((SPLIT * NB1,), jnp.int32),
        pltpu.VMEM((L,), jnp.int32),
    ],
    compiler_params=pltpu.CompilerParams(needs_layout_passes=False),
)(_thr_body)


BR, BC = 8, 32768


def _mask_body(thr_ref, x_ref, y_ref, m_ref):
    xb = x_ref[...]
    keep = jnp.abs(xb) >= thr_ref[...]
    y_ref[...] = jnp.where(keep, xb, 0.0)
    m_ref[...] = keep.astype(jnp.float32)


_apply_mask = pl.pallas_call(
    _mask_body,
    grid=(N_ROWS // BR, N_COLS // BC),
    in_specs=[
        pl.BlockSpec((BR, 1), lambda i, j: (i, 0)),
        pl.BlockSpec((BR, BC), lambda i, j: (i, j)),
    ],
    out_specs=[
        pl.BlockSpec((BR, BC), lambda i, j: (i, j)),
        pl.BlockSpec((BR, BC), lambda i, j: (i, j)),
    ],
    out_shape=[
        jax.ShapeDtypeStruct((N_ROWS, N_COLS), jnp.float32),
        jax.ShapeDtypeStruct((N_ROWS, N_COLS), jnp.float32),
    ],
)


@jax.jit
def kernel(x):
    xi = lax.bitcast_convert_type(x.reshape(-1), jnp.int32)
    thr_flat = _sc_thresholds(xi)
    thr_bits = thr_flat.reshape(NW, L)[:, :ROWS_PER_W].reshape(N_ROWS, 1)
    thr = lax.bitcast_convert_type(thr_bits, jnp.float32)
    y, m = _apply_mask(thr, x)
    return y, m


# speedup vs baseline: 16.9059x; 1.0299x over previous
import functools

import jax
import jax.numpy as jnp
from jax import lax
from jax.experimental import pallas as pl
from jax.experimental.pallas import tpu as pltpu
from jax.experimental.pallas import tpu_sc as plsc

N_ROWS = 128
N_COLS = 32768
K_RANK = N_COLS // 2
L = 16
NC, NS = 2, 16
NW = NC * NS
ROWS_PER_W = N_ROWS // NW
NV = N_COLS // L

B1_BITS, B2_BITS, B3_BITS = 11, 10, 10
NB1, NB2, NB3 = 1 << B1_BITS, 1 << B2_BITS, 1 << B3_BITS
SPLIT = 8
SIGN_MASK = 0x7FFFFFFF


def _bcast(s):
    return lax.broadcast_in_dim(s, (L,), ())


def _thr_body(x_hbm, thr_hbm, xrow, hist, tbuf):
    c = lax.axis_index("c")
    s = lax.axis_index("s")
    wid = s * NC + c

    lane = lax.broadcasted_iota(jnp.int32, (L,), 0)
    zeros_i = jnp.zeros((L,), jnp.int32)
    ones_i = jnp.ones((L,), jnp.int32)

    @plsc.parallel_loop(0, SPLIT * NB1 // L, unroll=8)
    def zbody(i):
        hist[pl.ds(i * L, L)] = zeros_i

    def magnitude(i):
        raw = xrow[pl.ds(i * L, L)]
        u = raw & SIGN_MASK
        return raw, u

    def find_bin(nbins, kprime):
        @plsc.parallel_loop(
            0, nbins // L, unroll=2,
            carry=(jnp.int32(0), jnp.int32(0), jnp.int32(0)),
        )
        def mcarry(j, carry):
            total, nless, cbefore = carry
            acc = hist[pl.ds(j * L, L)]
            hist[pl.ds(j * L, L)] = zeros_i
            for sub in range(1, SPLIT):
                off = sub * nbins + j * L
                acc = acc + hist[pl.ds(off, L)]
                hist[pl.ds(off, L)] = zeros_i
            cum = jnp.cumsum(acc) + _bcast(total)
            mlt = cum < _bcast(kprime)
            nless = nless + jnp.sum(jnp.where(mlt, ones_i, zeros_i))
            cbefore = jnp.maximum(cbefore, jnp.max(jnp.where(mlt, cum, zeros_i)))
            total = jnp.max(cum)
            return total, nless, cbefore

        _, nless, cbefore = mcarry
        return nless, cbefore

    def do_row(r, thrvec):
        row_base = (wid * ROWS_PER_W + r) * N_COLS
        pltpu.sync_copy(x_hbm.at[pl.ds(row_base, N_COLS)], xrow)

        @plsc.parallel_loop(0, NV, unroll=8)
        def s1(i):
            _, u = magnitude(i)
            b = lax.shift_right_logical(u, B2_BITS + B3_BITS)
            plsc.addupdate_scatter(hist, [(i & (SPLIT - 1)) * NB1 + b], ones_i)

        kprime = jnp.int32(K_RANK)
        b1, cbefore = find_bin(NB1, kprime)
        kprime = kprime - cbefore

        b1v = _bcast(b1)

        @plsc.parallel_loop(0, NV, unroll=8)
        def s2(i):
            _, u = magnitude(i)
            p = lax.shift_right_logical(u, B2_BITS + B3_BITS)
            b = lax.shift_right_logical(u, B3_BITS) & (NB2 - 1)
            plsc.addupdate_scatter(
                hist, [(i & (SPLIT - 1)) * NB2 + b], ones_i, mask=p == b1v)

        b2, cbefore = find_bin(NB2, kprime)
        kprime = kprime - cbefore

        prefix2 = (b1 << B2_BITS) | b2
        p2v = _bcast(prefix2)

        @plsc.parallel_loop(0, NV, unroll=8)
        def s3(i):
            _, u = magnitude(i)
            p = lax.shift_right_logical(u, B3_BITS)
            b = u & (NB3 - 1)
            plsc.addupdate_scatter(
                hist, [(i & (SPLIT - 1)) * NB3 + b], ones_i, mask=p == p2v)

        b3, _ = find_bin(NB3, kprime)

        thr = (prefix2 << B3_BITS) | b3
        return jnp.where(lane == _bcast(r), _bcast(thr), thrvec)

    thrvec = lax.fori_loop(0, ROWS_PER_W, do_row, zeros_i)
    tbuf[...] = thrvec
    pltpu.sync_copy(tbuf, thr_hbm.at[pl.ds(wid * L, L)])


_sc_thresholds = functools.partial(
    pl.kernel,
    out_type=jax.ShapeDtypeStruct((NW * L,), jnp.int32),
    mesh=plsc.VectorSubcoreMesh(
        core_axis_name="c", subcore_axis_name="s", num_cores=NC, num_subcores=NS
    ),
    scratch_types=[
        pltpu.VMEM((N_COLS,), jnp.int32),
        pltpu.VMEM((SPLIT * NB1,), jnp.int32),
        pltpu.VMEM((L,), jnp.int32),
    ],
    compiler_params=pltpu.CompilerParams(needs_layout_passes=False),
)(_thr_body)


BR, BC = 16, 32768


def _mask_body(thr_ref, x_ref, y_ref, m_ref):
    xb = x_ref[...]
    keep = jnp.abs(xb) >= thr_ref[...]
    y_ref[...] = jnp.where(keep, xb, 0.0)
    m_ref[...] = keep.astype(jnp.float32)


_apply_mask = pl.pallas_call(
    _mask_body,
    grid=(N_ROWS // BR, N_COLS // BC),
    in_specs=[
        pl.BlockSpec((BR, 1), lambda i, j: (i, 0)),
        pl.BlockSpec((BR, BC), lambda i, j: (i, j)),
    ],
    out_specs=[
        pl.BlockSpec((BR, BC), lambda i, j: (i, j)),
        pl.BlockSpec((BR, BC), lambda i, j: (i, j)),
    ],
    out_shape=[
        jax.ShapeDtypeStruct((N_ROWS, N_COLS), jnp.float32),
        jax.ShapeDtypeStruct((N_ROWS, N_COLS), jnp.float32),
    ],
)


@jax.jit
def kernel(x):
    xi = lax.bitcast_convert_type(x.reshape(-1), jnp.int32)
    thr_flat = _sc_thresholds(xi)
    thr_bits = thr_flat.reshape(NW, L)[:, :ROWS_PER_W].reshape(N_ROWS, 1)
    thr = lax.bitcast_convert_type(thr_bits, jnp.float32)
    y, m = _apply_mask(thr, x)
    return y, m


# speedup vs baseline: 17.4676x; 1.0332x over previous
import functools

import jax
import jax.numpy as jnp
from jax import lax
from jax.experimental import pallas as pl
from jax.experimental.pallas import tpu as pltpu
from jax.experimental.pallas import tpu_sc as plsc

N_ROWS = 128
N_COLS = 32768
K_RANK = N_COLS // 2
L = 16
NC, NS = 2, 16
NW = NC * NS
ROWS_PER_W = N_ROWS // NW
NV = N_COLS // L

B1_BITS, B2_BITS, B3_BITS = 11, 10, 10
NB1, NB2, NB3 = 1 << B1_BITS, 1 << B2_BITS, 1 << B3_BITS
SIGN_MASK = 0x7FFFFFFF


def _bcast(s):
    return lax.broadcast_in_dim(s, (L,), ())


def _thr_body(x_hbm, thr_hbm, xbuf0, xbuf1, hist, tbuf, sem0, sem1):
    c = lax.axis_index("c")
    s = lax.axis_index("s")
    wid = s * NC + c

    lane = lax.broadcasted_iota(jnp.int32, (L,), 0)
    zeros_i = jnp.zeros((L,), jnp.int32)
    ones_i = jnp.ones((L,), jnp.int32)

    @plsc.parallel_loop(0, NB1 // L, unroll=8)
    def zbody(i):
        hist[pl.ds(i * L, L)] = zeros_i

    def magnitude(xrow, i):
        raw = xrow[pl.ds(i * L, L)]
        u = raw & SIGN_MASK
        return raw, u

    def find_bin(nbins, kprime):
        @plsc.parallel_loop(
            0, nbins // L, unroll=2,
            carry=(jnp.int32(0), jnp.int32(0), jnp.int32(0)),
        )
        def mcarry(j, carry):
            total, nless, cbefore = carry
            acc = hist[pl.ds(j * L, L)]
            hist[pl.ds(j * L, L)] = zeros_i
            cum = jnp.cumsum(acc) + _bcast(total)
            mlt = cum < _bcast(kprime)
            nless = nless + jnp.sum(jnp.where(mlt, ones_i, zeros_i))
            cbefore = jnp.maximum(cbefore, jnp.max(jnp.where(mlt, cum, zeros_i)))
            total = jnp.max(cum)
            return total, nless, cbefore

        _, nless, cbefore = mcarry
        return nless, cbefore

    sems = (sem0, sem1)
    bufs = (xbuf0, xbuf1)

    def row_dma(r):
        row_base = (wid * ROWS_PER_W + r) * N_COLS
        return pltpu.async_copy(
            x_hbm.at[pl.ds(row_base, N_COLS)], bufs[r % 2], sems[r % 2])

    def do_row(r, xrow, thrvec):
        @plsc.parallel_loop(0, NV, unroll=8)
        def s1(i):
            _, u = magnitude(xrow, i)
            b = lax.shift_right_logical(u, B2_BITS + B3_BITS)
            plsc.addupdate_scatter(hist, [b], ones_i)

        kprime = jnp.int32(K_RANK)
        b1, cbefore = find_bin(NB1, kprime)
        kprime = kprime - cbefore

        b1v = _bcast(b1)

        @plsc.parallel_loop(0, NV, unroll=8)
        def s2(i):
            _, u = magnitude(xrow, i)
            p = lax.shift_right_logical(u, B2_BITS + B3_BITS)
            b = lax.shift_right_logical(u, B3_BITS) & (NB2 - 1)
            plsc.addupdate_scatter(hist, [b], ones_i, mask=p == b1v)

        b2, cbefore = find_bin(NB2, kprime)
        kprime = kprime - cbefore

        prefix2 = (b1 << B2_BITS) | b2
        p2v = _bcast(prefix2)

        @plsc.parallel_loop(0, NV, unroll=8)
        def s3(i):
            _, u = magnitude(xrow, i)
            p = lax.shift_right_logical(u, B3_BITS)
            b = u & (NB3 - 1)
            plsc.addupdate_scatter(hist, [b], ones_i, mask=p == p2v)

        b3, _ = find_bin(NB3, kprime)

        thr = (prefix2 << B3_BITS) | b3
        return jnp.where(lane == _bcast(jnp.int32(r)), _bcast(thr), thrvec)

    thrvec = zeros_i
    pending = row_dma(0)
    for r in range(ROWS_PER_W):
        pending.wait()
        if r + 1 < ROWS_PER_W:
            pending = row_dma(r + 1)
        thrvec = do_row(r, bufs[r % 2], thrvec)
    tbuf[...] = thrvec
    pltpu.sync_copy(tbuf, thr_hbm.at[pl.ds(wid * L, L)])


_sc_thresholds = functools.partial(
    pl.kernel,
    out_type=jax.ShapeDtypeStruct((NW * L,), jnp.int32),
    mesh=plsc.VectorSubcoreMesh(
        core_axis_name="c", subcore_axis_name="s", num_cores=NC, num_subcores=NS
    ),
    scratch_types=[
        pltpu.VMEM((N_COLS,), jnp.int32),
        pltpu.VMEM((N_COLS,), jnp.int32),
        pltpu.VMEM((NB1,), jnp.int32),
        pltpu.VMEM((L,), jnp.int32),
        pltpu.SemaphoreType.DMA,
        pltpu.SemaphoreType.DMA,
    ],
    compiler_params=pltpu.CompilerParams(needs_layout_passes=False),
)(_thr_body)


BR, BC = 16, 32768


def _mask_body(thr_ref, x_ref, y_ref, m_ref):
    xb = x_ref[...]
    keep = jnp.abs(xb) >= thr_ref[...]
    y_ref[...] = jnp.where(keep, xb, 0.0)
    m_ref[...] = keep.astype(jnp.float32)


_apply_mask = pl.pallas_call(
    _mask_body,
    grid=(N_ROWS // BR, N_COLS // BC),
    in_specs=[
        pl.BlockSpec((BR, 1), lambda i, j: (i, 0)),
        pl.BlockSpec((BR, BC), lambda i, j: (i, j)),
    ],
    out_specs=[
        pl.BlockSpec((BR, BC), lambda i, j: (i, j)),
        pl.BlockSpec((BR, BC), lambda i, j: (i, j)),
    ],
    out_shape=[
        jax.ShapeDtypeStruct((N_ROWS, N_COLS), jnp.float32),
        jax.ShapeDtypeStruct((N_ROWS, N_COLS), jnp.float32),
    ],
)


@jax.jit
def kernel(x):
    xi = lax.bitcast_convert_type(x.reshape(-1), jnp.int32)
    thr_flat = _sc_thresholds(xi)
    thr_bits = thr_flat.reshape(NW, L)[:, :ROWS_PER_W].reshape(N_ROWS, 1)
    thr = lax.bitcast_convert_type(thr_bits, jnp.float32)
    y, m = _apply_mask(thr, x)
    return y, m


# speedup vs baseline: 17.8559x; 1.0222x over previous
import functools

import jax
import jax.numpy as jnp
from jax import lax
from jax.experimental import pallas as pl
from jax.experimental.pallas import tpu as pltpu
from jax.experimental.pallas import tpu_sc as plsc

N_ROWS = 128
N_COLS = 32768
K_RANK = N_COLS // 2
L = 16
NC, NS = 2, 16
NW = NC * NS
ROWS_PER_W = N_ROWS // NW
NV = N_COLS // L

B1_BITS, B2_BITS, B3_BITS = 11, 10, 10
NB1, NB2, NB3 = 1 << B1_BITS, 1 << B2_BITS, 1 << B3_BITS
SIGN_MASK = 0x7FFFFFFF


def _bcast(s):
    return lax.broadcast_in_dim(s, (L,), ())


def _thr_body(x_hbm, thr_hbm, xbuf0, xbuf1, hist, cbuf, tbuf, sem0, sem1):
    c = lax.axis_index("c")
    s = lax.axis_index("s")
    wid = s * NC + c

    lane = lax.broadcasted_iota(jnp.int32, (L,), 0)
    zeros_i = jnp.zeros((L,), jnp.int32)
    ones_i = jnp.ones((L,), jnp.int32)

    @plsc.parallel_loop(0, NB1 // L, unroll=8)
    def zbody(i):
        hist[pl.ds(i * L, L)] = zeros_i

    def magnitude(xrow, i):
        raw = xrow[pl.ds(i * L, L)]
        u = raw & SIGN_MASK
        return raw, u

    def find_bin(nbins, kprime):
        @plsc.parallel_loop(
            0, nbins // L, unroll=2,
            carry=(jnp.int32(0), jnp.int32(0), jnp.int32(0)),
        )
        def mcarry(j, carry):
            total, nless, cbefore = carry
            acc = hist[pl.ds(j * L, L)]
            hist[pl.ds(j * L, L)] = zeros_i
            cum = jnp.cumsum(acc) + _bcast(total)
            mlt = cum < _bcast(kprime)
            nless = nless + jnp.sum(jnp.where(mlt, ones_i, zeros_i))
            cbefore = jnp.maximum(cbefore, jnp.max(jnp.where(mlt, cum, zeros_i)))
            total = jnp.max(cum)
            return total, nless, cbefore

        _, nless, cbefore = mcarry
        return nless, cbefore

    sems = (sem0, sem1)
    bufs = (xbuf0, xbuf1)

    def row_dma(r):
        row_base = (wid * ROWS_PER_W + r) * N_COLS
        return pltpu.async_copy(
            x_hbm.at[pl.ds(row_base, N_COLS)], bufs[r % 2], sems[r % 2])

    def do_row(r, xrow, thrvec):
        @plsc.parallel_loop(0, NV, unroll=8)
        def s1(i):
            _, u = magnitude(xrow, i)
            b = lax.shift_right_logical(u, B2_BITS + B3_BITS)
            plsc.addupdate_scatter(hist, [b], ones_i)

        kprime = jnp.int32(K_RANK)
        b1, cbefore = find_bin(NB1, kprime)
        kprime = kprime - cbefore

        b1v = _bcast(b1)

        @plsc.parallel_loop(0, NV, unroll=8, carry=jnp.zeros((L,), jnp.int32))
        def s2(i, posv):
            _, u = magnitude(xrow, i)
            p = lax.shift_right_logical(u, B2_BITS + B3_BITS)
            m = p == b1v
            b = lax.shift_right_logical(u, B3_BITS) & (NB2 - 1)
            plsc.addupdate_scatter(hist, [b], ones_i, mask=m)
            plsc.store_compressed(cbuf.at[pl.ds(posv[0], L)], u, mask=m)
            return posv + plsc.all_reduce_population_count(m)

        n2 = s2[0]
        b2, cbefore = find_bin(NB2, kprime)
        kprime = kprime - cbefore

        prefix2 = (b1 << B2_BITS) | b2
        p2v = _bcast(prefix2)
        n2v = _bcast(n2)
        nv3 = lax.shift_right_logical(n2 + (L - 1), 4)

        @plsc.parallel_loop(0, nv3)
        def s3(i):
            u = cbuf[pl.ds(i * L, L)]
            valid = (_bcast(i * L) + lane) < n2v
            p = lax.shift_right_logical(u, B3_BITS)
            b = u & (NB3 - 1)
            plsc.addupdate_scatter(hist, [b], ones_i, mask=valid & (p == p2v))

        b3, _ = find_bin(NB3, kprime)

        thr = (prefix2 << B3_BITS) | b3
        return jnp.where(lane == _bcast(jnp.int32(r)), _bcast(thr), thrvec)

    thrvec = zeros_i
    pending = row_dma(0)
    for r in range(ROWS_PER_W):
        pending.wait()
        if r + 1 < ROWS_PER_W:
            pending = row_dma(r + 1)
        thrvec = do_row(r, bufs[r % 2], thrvec)
    tbuf[...] = thrvec
    pltpu.sync_copy(tbuf, thr_hbm.at[pl.ds(wid * L, L)])


_sc_thresholds = functools.partial(
    pl.kernel,
    out_type=jax.ShapeDtypeStruct((NW * L,), jnp.int32),
    mesh=plsc.VectorSubcoreMesh(
        core_axis_name="c", subcore_axis_name="s", num_cores=NC, num_subcores=NS
    ),
    scratch_types=[
        pltpu.VMEM((N_COLS,), jnp.int32),
        pltpu.VMEM((N_COLS,), jnp.int32),
        pltpu.VMEM((NB1,), jnp.int32),
        pltpu.VMEM((N_COLS,), jnp.int32),
        pltpu.VMEM((L,), jnp.int32),
        pltpu.SemaphoreType.DMA,
        pltpu.SemaphoreType.DMA,
    ],
    compiler_params=pltpu.CompilerParams(needs_layout_passes=False),
)(_thr_body)


BR, BC = 16, 32768


def _mask_body(thr_ref, x_ref, y_ref, m_ref):
    xb = x_ref[...]
    keep = jnp.abs(xb) >= thr_ref[...]
    y_ref[...] = jnp.where(keep, xb, 0.0)
    m_ref[...] = keep.astype(jnp.float32)


_apply_mask = pl.pallas_call(
    _mask_body,
    grid=(N_ROWS // BR, N_COLS // BC),
    in_specs=[
        pl.BlockSpec((BR, 1), lambda i, j: (i, 0)),
        pl.BlockSpec((BR, BC), lambda i, j: (i, j)),
    ],
    out_specs=[
        pl.BlockSpec((BR, BC), lambda i, j: (i, j)),
        pl.BlockSpec((BR, BC), lambda i, j: (i, j)),
    ],
    out_shape=[
        jax.ShapeDtypeStruct((N_ROWS, N_COLS), jnp.float32),
        jax.ShapeDtypeStruct((N_ROWS, N_COLS), jnp.float32),
    ],
)


@jax.jit
def kernel(x):
    xi = lax.bitcast_convert_type(x.reshape(-1), jnp.int32)
    thr_flat = _sc_thresholds(xi)
    thr_bits = thr_flat.reshape(NW, L)[:, :ROWS_PER_W].reshape(N_ROWS, 1)
    thr = lax.bitcast_convert_type(thr_bits, jnp.float32)
    y, m = _apply_mask(thr, x)
    return y, m


# speedup vs baseline: 18.1942x; 1.0189x over previous
import functools

import jax
import jax.numpy as jnp
from jax import lax
from jax.experimental import pallas as pl
from jax.experimental.pallas import tpu as pltpu
from jax.experimental.pallas import tpu_sc as plsc

N_ROWS = 128
N_COLS = 32768
K_RANK = N_COLS // 2
L = 16
NC, NS = 2, 16
NW = NC * NS
ROWS_PER_W = N_ROWS // NW
NV = N_COLS // L

B1_BITS, B2_BITS, B3_BITS = 11, 10, 10
NB1, NB2, NB3 = 1 << B1_BITS, 1 << B2_BITS, 1 << B3_BITS
SIGN_MASK = 0x7FFFFFFF


def _bcast(s):
    return lax.broadcast_in_dim(s, (L,), ())


def _thr_body(x_hbm, thr_hbm, xbuf0, xbuf1, hist, cbuf, tbuf, sem0, sem1):
    c = lax.axis_index("c")
    s = lax.axis_index("s")
    wid = s * NC + c

    lane = lax.broadcasted_iota(jnp.int32, (L,), 0)
    zeros_i = jnp.zeros((L,), jnp.int32)
    ones_i = jnp.ones((L,), jnp.int32)

    @plsc.parallel_loop(0, NB1 // L, unroll=8)
    def zbody(i):
        hist[pl.ds(i * L, L)] = zeros_i

    def magnitude(xrow, i):
        raw = xrow[pl.ds(i * L, L)]
        u = raw & SIGN_MASK
        return raw, u

    def find_bin(nbins, kprime):
        @plsc.parallel_loop(
            0, nbins // L, unroll=2,
            carry=(jnp.int32(0), jnp.int32(0), jnp.int32(0)),
        )
        def mcarry(j, carry):
            total, nless, cbefore = carry
            acc = hist[pl.ds(j * L, L)]
            hist[pl.ds(j * L, L)] = zeros_i
            cum = jnp.cumsum(acc) + _bcast(total)
            mlt = cum < _bcast(kprime)
            nless = nless + jnp.sum(jnp.where(mlt, ones_i, zeros_i))
            cbefore = jnp.maximum(cbefore, jnp.max(jnp.where(mlt, cum, zeros_i)))
            total = jnp.max(cum)
            return total, nless, cbefore

        _, nless, cbefore = mcarry
        return nless, cbefore

    sems = (sem0, sem1)
    bufs = (xbuf0, xbuf1)

    def row_dma(r):
        row_base = (wid * ROWS_PER_W + r) * N_COLS
        return pltpu.async_copy(
            x_hbm.at[pl.ds(row_base, N_COLS)], bufs[r % 2], sems[r % 2])

    def do_row(r, xrow, thrvec):
        @plsc.parallel_loop(0, NV, unroll=8)
        def s1(i):
            _, u = magnitude(xrow, i)
            b = lax.shift_right_logical(u, B2_BITS + B3_BITS)
            plsc.addupdate_scatter(hist, [b], ones_i)

        kprime = jnp.int32(K_RANK)
        b1, cbefore = find_bin(NB1, kprime)
        kprime = kprime - cbefore

        b1v = _bcast(b1)

        @plsc.parallel_loop(0, NV, unroll=8, carry=jnp.zeros((L,), jnp.int32))
        def s2(i, posv):
            _, u = magnitude(xrow, i)
            p = lax.shift_right_logical(u, B2_BITS + B3_BITS)
            m = p == b1v
            plsc.store_compressed(cbuf.at[pl.ds(posv[0], L)], u, mask=m)
            return posv + plsc.all_reduce_population_count(m)

        n2 = s2[0]
        n2v = _bcast(n2)
        nv2 = lax.shift_right_logical(n2 + (L - 1), 4)

        @plsc.parallel_loop(0, nv2)
        def s2b(i):
            u = cbuf[pl.ds(i * L, L)]
            valid = (_bcast(i * L) + lane) < n2v
            b = lax.shift_right_logical(u, B3_BITS) & (NB2 - 1)
            plsc.addupdate_scatter(hist, [b], ones_i, mask=valid)

        b2, cbefore = find_bin(NB2, kprime)
        kprime = kprime - cbefore

        prefix2 = (b1 << B2_BITS) | b2
        p2v = _bcast(prefix2)

        @plsc.parallel_loop(0, nv2)
        def s3(i):
            u = cbuf[pl.ds(i * L, L)]
            valid = (_bcast(i * L) + lane) < n2v
            p = lax.shift_right_logical(u, B3_BITS)
            b = u & (NB3 - 1)
            plsc.addupdate_scatter(hist, [b], ones_i, mask=valid & (p == p2v))

        b3, _ = find_bin(NB3, kprime)

        thr = (prefix2 << B3_BITS) | b3
        return jnp.where(lane == _bcast(jnp.int32(r)), _bcast(thr), thrvec)

    thrvec = zeros_i
    pending = row_dma(0)
    for r in range(ROWS_PER_W):
        pending.wait()
        if r + 1 < ROWS_PER_W:
            pending = row_dma(r + 1)
        thrvec = do_row(r, bufs[r % 2], thrvec)
    tbuf[...] = thrvec
    pltpu.sync_copy(tbuf, thr_hbm.at[pl.ds(wid * L, L)])


_sc_thresholds = functools.partial(
    pl.kernel,
    out_type=jax.ShapeDtypeStruct((NW * L,), jnp.int32),
    mesh=plsc.VectorSubcoreMesh(
        core_axis_name="c", subcore_axis_name="s", num_cores=NC, num_subcores=NS
    ),
    scratch_types=[
        pltpu.VMEM((N_COLS,), jnp.int32),
        pltpu.VMEM((N_COLS,), jnp.int32),
        pltpu.VMEM((NB1,), jnp.int32),
        pltpu.VMEM((N_COLS,), jnp.int32),
        pltpu.VMEM((L,), jnp.int32),
        pltpu.SemaphoreType.DMA,
        pltpu.SemaphoreType.DMA,
    ],
    compiler_params=pltpu.CompilerParams(needs_layout_passes=False),
)(_thr_body)


BR, BC = 16, 32768


def _mask_body(thr_ref, x_ref, y_ref, m_ref):
    xb = x_ref[...]
    keep = jnp.abs(xb) >= thr_ref[...]
    y_ref[...] = jnp.where(keep, xb, 0.0)
    m_ref[...] = keep.astype(jnp.float32)


_apply_mask = pl.pallas_call(
    _mask_body,
    grid=(N_ROWS // BR, N_COLS // BC),
    in_specs=[
        pl.BlockSpec((BR, 1), lambda i, j: (i, 0)),
        pl.BlockSpec((BR, BC), lambda i, j: (i, j)),
    ],
    out_specs=[
        pl.BlockSpec((BR, BC), lambda i, j: (i, j)),
        pl.BlockSpec((BR, BC), lambda i, j: (i, j)),
    ],
    out_shape=[
        jax.ShapeDtypeStruct((N_ROWS, N_COLS), jnp.float32),
        jax.ShapeDtypeStruct((N_ROWS, N_COLS), jnp.float32),
    ],
)


@jax.jit
def kernel(x):
    xi = lax.bitcast_convert_type(x.reshape(-1), jnp.int32)
    thr_flat = _sc_thresholds(xi)
    thr_bits = thr_flat.reshape(NW, L)[:, :ROWS_PER_W].reshape(N_ROWS, 1)
    thr = lax.bitcast_convert_type(thr_bits, jnp.float32)
    y, m = _apply_mask(thr, x)
    return y, m
